# baseline (speedup 1.0000x reference)
import functools

import jax
import jax.numpy as jnp
from jax import lax
from jax.experimental import pallas as pl
from jax.experimental.pallas import tpu as pltpu
from jax.experimental.pallas import tpu_sc as plsc

_EPS = 1e-12
_NC = 2
_NS = 16
_NW = _NC * _NS


def _pick_chunk(n_per_w):
    for c in range(128, 0, -8):
        if n_per_w % c == 0:
            return c
    raise ValueError(n_per_w)


def _sc_gather(table, idx):
    n = idx.shape[0]
    d = table.shape[1]
    n_per_w = n // _NW
    chunk = _pick_chunk(n_per_w)
    n_chunks = n_per_w // chunk
    idx3 = idx.reshape(_NW, n_chunks, chunk)

    mesh = plsc.VectorSubcoreMesh(
        core_axis_name="c", subcore_axis_name="s",
        num_cores=_NC, num_subcores=_NS)

    @functools.partial(
        pl.kernel,
        out_type=jax.ShapeDtypeStruct((n, d), jnp.float32),
        mesh=mesh,
        scratch_types=[
            pltpu.VMEM((n_chunks, chunk), jnp.int32),
            pltpu.VMEM((chunk, d), jnp.float32),
            pltpu.VMEM((chunk, d), jnp.float32),
            pltpu.SemaphoreType.DMA,
            pltpu.SemaphoreType.DMA,
        ],
        compiler_params=pltpu.CompilerParams(use_tc_tiling_on_sc=True),
    )
    def gather_kernel(table_hbm, idx_hbm, out_hbm, idx_v, buf0, buf1, sem0, sem1):
        wid = lax.axis_index("s") * _NC + lax.axis_index("c")
        pltpu.sync_copy(idx_hbm.at[wid], idx_v)
        bufs = (buf0, buf1)
        sems = (sem0, sem1)
        base = wid * n_per_w
        cps = [None] * n_chunks
        cps[0] = pltpu.async_copy(table_hbm.at[idx_v.at[0]], bufs[0], sems[0])
        for j in range(n_chunks):
            if j + 1 < n_chunks:
                cps[j + 1] = pltpu.async_copy(
                    table_hbm.at[idx_v.at[j + 1]], bufs[(j + 1) % 2], sems[(j + 1) % 2])
            cps[j].wait()
            pltpu.sync_copy(bufs[j % 2], out_hbm.at[pl.ds(base + j * chunk, chunk)])

    return gather_kernel(table, idx3)


def _tc_work(maxpos, m, r, ent_ref, pid_ref, pos_ref, w_ref, g_ref, b_ref,
             out_ref):
    ent = ent_ref[...].astype(jnp.bfloat16)
    proj = jnp.dot(ent, w_ref[...], preferred_element_type=jnp.float32)
    pid = pid_ref[...]
    iota = lax.broadcasted_iota(jnp.int32, (r, maxpos), 1)
    counts = jnp.zeros((r, maxpos), jnp.bfloat16)
    for j in range(m):
        counts += (pid[:, j:j + 1] == iota).astype(jnp.bfloat16)
    possum = jnp.dot(counts, pos_ref[...], preferred_element_type=jnp.float32)
    denom = jnp.maximum(
        jnp.sum((pid != 0).astype(jnp.float32), axis=1, keepdims=True), 1.0)
    x = proj + possum * (1.0 / denom)
    mu = jnp.mean(x, axis=-1, keepdims=True)
    xc = x - mu
    var = jnp.mean(xc * xc, axis=-1, keepdims=True)
    y = xc * lax.rsqrt(var + _EPS) * g_ref[...] + b_ref[...]
    out_ref[...] = y.reshape(1, r, y.shape[-1])


def _tc_body(maxpos, m, r, ent_ref, pid_ref, pos_ref, w_ref, g_ref, b_ref,
             out_ref):
    _tc_work(maxpos, m, r, ent_ref, pid_ref, pos_ref, w_ref, g_ref, b_ref,
             out_ref)


def _tc_body_carry(maxpos, m, r, ent_ref, pid_ref, pos_ref, w_ref, g_ref,
                   b_ref, carry_ref, out_ref):
    del carry_ref
    _tc_work(maxpos, m, r, ent_ref, pid_ref, pos_ref, w_ref, g_ref, b_ref,
             out_ref)


def _tc_compute(gathered, pid, pos_table, dense_w, gamma2d, beta2d, b, l_out,
                l0, carry=None, *, interpret=False):
    n, emb = gathered.shape
    maxpos, hid = pos_table.shape
    m = pid.shape[1]
    r = b
    nl = n // b
    in_specs = [
        pl.BlockSpec((r, emb), lambda i: (i, 0)),
        pl.BlockSpec((r, m), lambda i: (i, 0)),
        pl.BlockSpec((maxpos, hid), lambda i: (0, 0)),
        pl.BlockSpec((emb, hid), lambda i: (0, 0)),
        pl.BlockSpec((1, hid), lambda i: (0, 0)),
        pl.BlockSpec((1, hid), lambda i: (0, 0)),
    ]
    args = [gathered, pid, pos_table, dense_w, gamma2d, beta2d]
    if carry is None:
        body = functools.partial(_tc_body, maxpos, m, r)
        aliases = {}
    else:
        body = functools.partial(_tc_body_carry, maxpos, m, r)
        in_specs.append(pl.BlockSpec(memory_space=pltpu.MemorySpace.HBM))
        args.append(carry)
        aliases = {6: 0}
    return pl.pallas_call(
        body,
        grid=(nl,),
        in_specs=in_specs,
        out_specs=pl.BlockSpec((1, r, hid), lambda i: (i + l0, 0, 0)),
        out_shape=jax.ShapeDtypeStruct((l_out, b, hid), jnp.float32),
        input_output_aliases=aliases,
        interpret=interpret,
    )(*args)


def kernel(entity_ids, entity_position_ids, entity_table, pos_table, dense_w,
           ln_gamma, ln_beta):
    b, l = entity_ids.shape
    m = entity_position_ids.shape[-1]
    hid = pos_table.shape[1]
    n = b * l
    splits = [(0, 4), (4, 12), (12, 20)] if l == 20 else [(0, l)]
    ids_t = entity_ids.T.astype(jnp.int32)
    pid3 = entity_position_ids.transpose(1, 0, 2).astype(jnp.int32)
    pos_bf = pos_table.astype(jnp.bfloat16)
    w_bf = dense_w.astype(jnp.bfloat16)
    g2 = ln_gamma.reshape(1, hid)
    b2 = ln_beta.reshape(1, hid)

    gathered = [
        _sc_gather(entity_table, ids_t[l0:l1].reshape((l1 - l0) * b))
        for l0, l1 in splits
    ]
    out = None
    for g, (l0, l1) in zip(gathered, splits):
        out = _tc_compute(g, pid3[l0:l1].reshape((l1 - l0) * b, m),
                          pos_bf, w_bf, g2, b2, b, l, l0, carry=out)
    return out.transpose(1, 0, 2)


# speedup vs baseline: 1.0427x; 1.0427x over previous
import functools

import jax
import jax.numpy as jnp
from jax import lax
from jax.experimental import pallas as pl
from jax.experimental.pallas import tpu as pltpu
from jax.experimental.pallas import tpu_sc as plsc

_EPS = 1e-12
_NC = 2
_NS = 16
_NW = _NC * _NS


def _pick_chunk(n_per_w):
    for c in range(128, 0, -8):
        if n_per_w % c == 0:
            return c
    raise ValueError(n_per_w)


def _sc_gather(table, idx):
    n = idx.shape[0]
    d = table.shape[1]
    n_per_w = n // _NW
    chunk = _pick_chunk(n_per_w)
    n_chunks = n_per_w // chunk
    idx3 = idx.reshape(_NW, n_chunks, chunk)

    mesh = plsc.VectorSubcoreMesh(
        core_axis_name="c", subcore_axis_name="s",
        num_cores=_NC, num_subcores=_NS)

    @functools.partial(
        pl.kernel,
        out_type=jax.ShapeDtypeStruct((n, d), jnp.float32),
        mesh=mesh,
        scratch_types=[
            pltpu.VMEM((n_chunks, chunk), jnp.int32),
            pltpu.VMEM((chunk, d), jnp.float32),
            pltpu.VMEM((chunk, d), jnp.float32),
            pltpu.SemaphoreType.DMA,
            pltpu.SemaphoreType.DMA,
        ],
        compiler_params=pltpu.CompilerParams(use_tc_tiling_on_sc=True),
    )
    def gather_kernel(table_hbm, idx_hbm, out_hbm, idx_v, buf0, buf1, sem0, sem1):
        wid = lax.axis_index("s") * _NC + lax.axis_index("c")
        pltpu.sync_copy(idx_hbm.at[wid], idx_v)
        bufs = (buf0, buf1)
        sems = (sem0, sem1)
        base = wid * n_per_w
        cps = [None] * n_chunks
        cps[0] = pltpu.async_copy(table_hbm.at[idx_v.at[0]], bufs[0], sems[0])
        for j in range(n_chunks):
            if j + 1 < n_chunks:
                cps[j + 1] = pltpu.async_copy(
                    table_hbm.at[idx_v.at[j + 1]], bufs[(j + 1) % 2], sems[(j + 1) % 2])
            cps[j].wait()
            pltpu.sync_copy(bufs[j % 2], out_hbm.at[pl.ds(base + j * chunk, chunk)])

    return gather_kernel(table, idx3)


def _tc_work(maxpos, m, r, ent_ref, pid_ref, pos_ref, w_ref, g_ref, b_ref,
             out_ref):
    ent = ent_ref[...].astype(jnp.bfloat16)
    proj = jnp.dot(ent, w_ref[...], preferred_element_type=jnp.float32)
    pid = pid_ref[...]
    iota = lax.broadcasted_iota(jnp.int32, (r, maxpos), 1)
    counts = jnp.zeros((r, maxpos), jnp.bfloat16)
    for j in range(m):
        counts += (pid[:, j:j + 1] == iota).astype(jnp.bfloat16)
    possum = jnp.dot(counts, pos_ref[...], preferred_element_type=jnp.float32)
    denom = jnp.maximum(
        jnp.sum((pid != 0).astype(jnp.float32), axis=1, keepdims=True), 1.0)
    x = proj + possum * (1.0 / denom)
    mu = jnp.mean(x, axis=-1, keepdims=True)
    xc = x - mu
    var = jnp.mean(xc * xc, axis=-1, keepdims=True)
    y = xc * lax.rsqrt(var + _EPS) * g_ref[...] + b_ref[...]
    out_ref[...] = y.reshape(1, r, y.shape[-1])


def _tc_body(maxpos, m, r, ent_ref, pid_ref, pos_ref, w_ref, g_ref, b_ref,
             out_ref):
    _tc_work(maxpos, m, r, ent_ref, pid_ref, pos_ref, w_ref, g_ref, b_ref,
             out_ref)


def _tc_body_carry(maxpos, m, r, ent_ref, pid_ref, pos_ref, w_ref, g_ref,
                   b_ref, carry_ref, out_ref):
    del carry_ref
    _tc_work(maxpos, m, r, ent_ref, pid_ref, pos_ref, w_ref, g_ref, b_ref,
             out_ref)


def _tc_compute(gathered, pid, pos_table, dense_w, gamma2d, beta2d, b, l_out,
                l0, carry=None, *, interpret=False):
    n, emb = gathered.shape
    maxpos, hid = pos_table.shape
    m = pid.shape[1]
    r = b
    nl = n // b
    in_specs = [
        pl.BlockSpec((r, emb), lambda i: (i, 0)),
        pl.BlockSpec((r, m), lambda i: (i, 0)),
        pl.BlockSpec((maxpos, hid), lambda i: (0, 0)),
        pl.BlockSpec((emb, hid), lambda i: (0, 0)),
        pl.BlockSpec((1, hid), lambda i: (0, 0)),
        pl.BlockSpec((1, hid), lambda i: (0, 0)),
    ]
    args = [gathered, pid, pos_table, dense_w, gamma2d, beta2d]
    if carry is None:
        body = functools.partial(_tc_body, maxpos, m, r)
        aliases = {}
    else:
        body = functools.partial(_tc_body_carry, maxpos, m, r)
        in_specs.append(pl.BlockSpec(memory_space=pltpu.MemorySpace.HBM))
        args.append(carry)
        aliases = {6: 0}
    return pl.pallas_call(
        body,
        grid=(nl,),
        in_specs=in_specs,
        out_specs=pl.BlockSpec((1, r, hid), lambda i: (i + l0, 0, 0)),
        out_shape=jax.ShapeDtypeStruct((l_out, b, hid), jnp.float32),
        input_output_aliases=aliases,
        interpret=interpret,
    )(*args)


def kernel(entity_ids, entity_position_ids, entity_table, pos_table, dense_w,
           ln_gamma, ln_beta):
    b, l = entity_ids.shape
    m = entity_position_ids.shape[-1]
    hid = pos_table.shape[1]
    n = b * l
    splits = [(0, l // 2), (l // 2, l)] if l % 2 == 0 else [(0, l)]
    ids_t = entity_ids.T.astype(jnp.int32)
    pid3 = entity_position_ids.transpose(1, 0, 2).astype(jnp.int32)
    pos_bf = pos_table.astype(jnp.bfloat16)
    w_bf = dense_w.astype(jnp.bfloat16)
    g2 = ln_gamma.reshape(1, hid)
    b2 = ln_beta.reshape(1, hid)

    gathered = [
        _sc_gather(entity_table, ids_t[l0:l1].reshape((l1 - l0) * b))
        for l0, l1 in splits
    ]
    out = None
    for g, (l0, l1) in zip(gathered, splits):
        out = _tc_compute(g, pid3[l0:l1].reshape((l1 - l0) * b, m),
                          pos_bf, w_bf, g2, b2, b, l, l0, carry=out)
    return out.transpose(1, 0, 2)
